# baseline (speedup 1.0000x reference)
import functools
import jax
import jax.numpy as jnp
from jax import lax
from jax.experimental import pallas as pl
from jax.experimental.pallas import tpu as pltpu, tpu_sc as plsc

NCH = 384
ROWS = 25088
NW = 32
RPW = ROWS // NW
R = 112
NCHUNK = RPW // R
NG = NCH // 16


def _make_sc_kernel():
    mesh = plsc.VectorSubcoreMesh(core_axis_name="c", subcore_axis_name="s")

    @functools.partial(
        pl.kernel,
        mesh=mesh,
        out_type=jax.ShapeDtypeStruct((ROWS, NCH), jnp.float32),
        scratch_types=[
            pltpu.VMEM((R, NCH), jnp.float32),
            pltpu.VMEM((R, NCH), jnp.float32),
            pltpu.SemaphoreType.DMA,
            pltpu.SemaphoreType.DMA,
            pltpu.SemaphoreType.DMA,
            pltpu.SemaphoreType.DMA,
        ],
    )
    def k(x_hbm, o_hbm, buf0, buf1, gsem0, gsem1, ssem0, ssem1):
        wid = lax.axis_index("s") * 2 + lax.axis_index("c")
        base = wid * RPW

        bufs = (buf0, buf1)
        gsems = (gsem0, gsem1)
        ssems = (ssem0, ssem1)

        def load(j):
            lo = pl.multiple_of(base + j * R, 8)
            return pltpu.make_async_copy(
                x_hbm.at[pl.ds(lo, R)], bufs[j % 2], gsems[j % 2]
            )

        def store(j):
            lo = pl.multiple_of(base + j * R, 8)
            return pltpu.make_async_copy(
                bufs[j % 2], o_hbm.at[pl.ds(lo, R)], ssems[j % 2]
            )

        def reverse_lanes(buf):
            def body(i, _):
                for g in range(NG // 2):
                    lo = pl.ds(16 * g, 16)
                    hi = pl.ds(NCH - 16 * (g + 1), 16)
                    t0 = buf[i, lo]
                    t1 = buf[i, hi]
                    buf[i, lo] = lax.rev(t1, (0,))
                    buf[i, hi] = lax.rev(t0, (0,))
                return _

            lax.fori_loop(0, R, body, None)

        load(0).start()
        for j in range(NCHUNK):
            if j + 1 < NCHUNK:
                if j >= 1:
                    store(j - 1).wait()
                load(j + 1).start()
            load(j).wait()
            reverse_lanes(bufs[j % 2])
            store(j).start()
        store(NCHUNK - 2).wait()
        store(NCHUNK - 1).wait()

    return k


_sc_kernel = _make_sc_kernel()


def kernel(input):
    b, c, h, w = input.shape
    xt = jnp.transpose(input, (0, 2, 3, 1)).reshape(b * h * w, c)
    out = _sc_kernel(xt)
    return jnp.transpose(out.reshape(b, h, w, c), (0, 3, 1, 2))


# speedup vs baseline: 1.0741x; 1.0741x over previous
import functools
import jax
import jax.numpy as jnp
from jax import lax
from jax.experimental import pallas as pl
from jax.experimental.pallas import tpu as pltpu, tpu_sc as plsc

NCH = 384
ROWS = 25088
NW = 32
RPW = ROWS // NW
R = 112
NCHUNK = RPW // R
NG = NCH // 16


def _make_sc_kernel():
    mesh = plsc.VectorSubcoreMesh(core_axis_name="c", subcore_axis_name="s")

    @functools.partial(
        pl.kernel,
        mesh=mesh,
        out_type=jax.ShapeDtypeStruct((ROWS, NCH), jnp.float32),
        scratch_types=[
            pltpu.VMEM((R, NCH), jnp.float32),
            pltpu.VMEM((R, NCH), jnp.float32),
            pltpu.SemaphoreType.DMA,
            pltpu.SemaphoreType.DMA,
            pltpu.SemaphoreType.DMA,
            pltpu.SemaphoreType.DMA,
        ],
    )
    def k(x_hbm, o_hbm, buf0, buf1, gsem0, gsem1, ssem0, ssem1):
        wid = lax.axis_index("s") * 2 + lax.axis_index("c")
        base = wid * RPW

        bufs = (buf0, buf1)
        gsems = (gsem0, gsem1)
        ssems = (ssem0, ssem1)

        def load(j):
            lo = pl.multiple_of(base + j * R, 8)
            return pltpu.make_async_copy(
                x_hbm.at[pl.ds(lo, R)], bufs[j % 2], gsems[j % 2]
            )

        def store(j):
            lo = pl.multiple_of(base + j * R, 8)
            return pltpu.make_async_copy(
                bufs[j % 2], o_hbm.at[pl.ds(lo, R)], ssems[j % 2]
            )

        col_rev = [
            jnp.full((16,), NCH - 1 - 16 * g, dtype=jnp.int32)
            - lax.iota(jnp.int32, 16)
            for g in range(NG)
        ]

        def reverse_lanes(buf):
            @functools.partial(plsc.parallel_loop, 0, R, unroll=2)
            def _loop(i):
                row = jnp.full((16,), i, dtype=jnp.int32)
                for g in range(NG // 2):
                    lo = pl.ds(16 * g, 16)
                    hi = pl.ds(NCH - 16 * (g + 1), 16)
                    a = plsc.load_gather(buf, [row, col_rev[g]])
                    b = plsc.load_gather(buf, [row, col_rev[NG - 1 - g]])
                    buf[i, lo] = a
                    buf[i, hi] = b

        load(0).start()
        for j in range(NCHUNK):
            if j + 1 < NCHUNK:
                if j >= 1:
                    store(j - 1).wait()
                load(j + 1).start()
            load(j).wait()
            reverse_lanes(bufs[j % 2])
            store(j).start()
        store(NCHUNK - 2).wait()
        store(NCHUNK - 1).wait()

    return k


_sc_kernel = _make_sc_kernel()


def kernel(input):
    b, c, h, w = input.shape
    xt = jnp.transpose(input, (0, 2, 3, 1)).reshape(b * h * w, c)
    out = _sc_kernel(xt)
    return jnp.transpose(out.reshape(b, h, w, c), (0, 3, 1, 2))
